# baseline (speedup 1.0000x reference)
import functools

import jax
import jax.numpy as jnp
from jax import lax
from jax.experimental import pallas as pl
from jax.experimental.pallas import tpu as pltpu
from jax.experimental.pallas import tpu_sc as plsc

N = 10000
E = 320000
D = 128
NC = 2
NS = 16
NW = NC * NS
CH = 96
NCH = 106
EPT = NCH * CH
EP = NW * EPT
NP = 10240
PADDST = NP - 8
RPS = NP // NS
ZR = 64

_mesh = plsc.VectorSubcoreMesh(core_axis_name="c", subcore_axis_name="s")
_sc_params = pltpu.CompilerParams(needs_layout_passes=False)

@functools.partial(
    pl.kernel,
    out_type=jax.ShapeDtypeStruct((NW * NP,), jnp.float32),
    mesh=_mesh,
    scratch_types=[
        pltpu.VMEM((NP,), jnp.float32),
        pltpu.VMEM((EPT,), jnp.int32),
    ],
    compiler_params=_sc_params,
)
def _deg_sc(dst_hbm, out_hbm, hist, idx):
    c = lax.axis_index("c")
    s = lax.axis_index("s")
    wid = c * NS + s
    zeros16 = jnp.zeros((16,), jnp.float32)
    ones16 = jnp.ones((16,), jnp.float32)

    def zbody(i, _):
        hist[pl.ds(i * 16, 16)] = zeros16
        return ()

    lax.fori_loop(0, NP // 16, zbody, ())
    pltpu.sync_copy(dst_hbm.at[pl.ds(wid * EPT, EPT)], idx)

    def body(i, _):
        iv = idx[pl.ds(i * 16, 16)]
        plsc.addupdate_scatter(hist, [iv], ones16)
        return ()

    lax.fori_loop(0, EPT // 16, body, ())
    pltpu.sync_copy(hist, out_hbm.at[pl.ds(wid * NP, NP)])


@functools.partial(
    pl.kernel,
    out_type=jax.ShapeDtypeStruct((NC, NP, D), jnp.float32),
    mesh=_mesh,
    scratch_types=[
        pltpu.VMEM((EPT,), jnp.int32),
        pltpu.VMEM((EPT,), jnp.int32),
        pltpu.VMEM((CH, D), jnp.float32),
        pltpu.VMEM((CH, D), jnp.float32),
        pltpu.VMEM_SHARED((NP, D), jnp.float32),
        pltpu.SemaphoreType.DMA,
        pltpu.SemaphoreType.DMA,
    ],
    compiler_params=_sc_params,
)
def _scatter_sc(src_hbm, dst_hbm, g_hbm, out_hbm,
                sidx, didx, rows_a, rows_b, acc, gsem_a, gsem_b):
    c = lax.axis_index("c")
    s = lax.axis_index("s")
    wid = c * NS + s
    pltpu.sync_copy(src_hbm.at[wid], sidx)
    pltpu.sync_copy(dst_hbm.at[wid], didx)
    pltpu.async_copy(g_hbm.at[sidx.at[pl.ds(0, CH)]], rows_a, gsem_a)

    zeros16 = jnp.zeros((16,), jnp.float32)

    def zrow(r, _):
        for jc in range(D // 16):
            rows_b[r, pl.ds(jc * 16, 16)] = zeros16
        return ()

    lax.fori_loop(0, ZR, zrow, ())
    for j in range(RPS // ZR):
        pltpu.sync_copy(rows_b.at[pl.ds(0, ZR)],
                        acc.at[pl.ds(s * RPS + j * ZR, ZR)])
    plsc.subcore_barrier()

    def body(i2, _):
        ia = 2 * i2
        ib = ia + 1
        pltpu.async_copy(g_hbm.at[sidx.at[pl.ds(ib * CH, CH)]], rows_b, gsem_b)
        pltpu.make_async_copy(g_hbm.at[sidx.at[pl.ds(ia * CH, CH)]], rows_a, gsem_a).wait()
        pltpu.sync_copy(rows_a, acc.at[didx.at[pl.ds(ia * CH, CH)]], add=True)
        nxt = ia + 2
        pltpu.async_copy(g_hbm.at[sidx.at[pl.ds(jnp.minimum(nxt, NCH - 2) * CH, CH)]], rows_a, gsem_a)
        pltpu.make_async_copy(g_hbm.at[sidx.at[pl.ds(ib * CH, CH)]], rows_b, gsem_b).wait()
        pltpu.sync_copy(rows_b, acc.at[didx.at[pl.ds(ib * CH, CH)]], add=True)
        return ()

    lax.fori_loop(0, NCH // 2 - 1, body, ())
    ia = NCH - 2
    pltpu.async_copy(g_hbm.at[sidx.at[pl.ds((NCH - 1) * CH, CH)]], rows_b, gsem_b)
    pltpu.make_async_copy(g_hbm.at[sidx.at[pl.ds(ia * CH, CH)]], rows_a, gsem_a).wait()
    pltpu.sync_copy(rows_a, acc.at[didx.at[pl.ds(ia * CH, CH)]], add=True)
    pltpu.make_async_copy(g_hbm.at[sidx.at[pl.ds((NCH - 1) * CH, CH)]], rows_b, gsem_b).wait()
    pltpu.sync_copy(rows_b, acc.at[didx.at[pl.ds((NCH - 1) * CH, CH)]], add=True)
    plsc.subcore_barrier()
    pltpu.sync_copy(acc.at[pl.ds(s * RPS, RPS)], out_hbm.at[c, pl.ds(s * RPS, RPS)])


BN = 400


def _dinv_body(p_ref, o_ref):
    deg = jnp.sum(p_ref[...], axis=0) + 1.0
    o_ref[...] = lax.rsqrt(deg)[:N, None]


_dinv_tc = pl.pallas_call(
    _dinv_body,
    out_shape=jax.ShapeDtypeStruct((N, 1), jnp.float32),
)


def _mm1_body(x_ref, w_ref, dv_ref, o_ref):
    h = jnp.dot(x_ref[...], w_ref[...], preferred_element_type=jnp.float32)
    o_ref[...] = h * dv_ref[...]


_mm1_tc = pl.pallas_call(
    _mm1_body,
    grid=(N // BN,),
    in_specs=[
        pl.BlockSpec((BN, D), lambda i: (i, 0)),
        pl.BlockSpec((D, D), lambda i: (0, 0)),
        pl.BlockSpec((BN, 1), lambda i: (i, 0)),
    ],
    out_specs=pl.BlockSpec((BN, D), lambda i: (i, 0)),
    out_shape=jax.ShapeDtypeStruct((N, D), jnp.float32),
)


def _mid_body(s_ref, g1_ref, dv_ref, b1_ref, w2_ref, o_ref):
    agg = (s_ref[0] + s_ref[1] + g1_ref[...]) * dv_ref[...] + b1_ref[...]
    h1 = jnp.maximum(agg, 0.0)
    h2 = jnp.dot(h1, w2_ref[...], preferred_element_type=jnp.float32)
    o_ref[...] = h2 * dv_ref[...]


_mid_tc = pl.pallas_call(
    _mid_body,
    grid=(N // BN,),
    in_specs=[
        pl.BlockSpec((NC, BN, D), lambda i: (0, i, 0)),
        pl.BlockSpec((BN, D), lambda i: (i, 0)),
        pl.BlockSpec((BN, 1), lambda i: (i, 0)),
        pl.BlockSpec((1, D), lambda i: (0, 0)),
        pl.BlockSpec((D, D), lambda i: (0, 0)),
    ],
    out_specs=pl.BlockSpec((BN, D), lambda i: (i, 0)),
    out_shape=jax.ShapeDtypeStruct((N, D), jnp.float32),
)


def _fin_body(s_ref, g2_ref, dv_ref, b2_ref, o_ref):
    o_ref[...] = (s_ref[0] + s_ref[1] + g2_ref[...]) * dv_ref[...] + b2_ref[...]


_fin_tc = pl.pallas_call(
    _fin_body,
    grid=(N // BN,),
    in_specs=[
        pl.BlockSpec((NC, BN, D), lambda i: (0, i, 0)),
        pl.BlockSpec((BN, D), lambda i: (i, 0)),
        pl.BlockSpec((BN, 1), lambda i: (i, 0)),
        pl.BlockSpec((1, D), lambda i: (0, 0)),
    ],
    out_specs=pl.BlockSpec((BN, D), lambda i: (i, 0)),
    out_shape=jax.ShapeDtypeStruct((N, D), jnp.float32),
)


def kernel(x, edge_index, W1, b1, W2, b2):
    src = edge_index[0].astype(jnp.int32)
    dst = edge_index[1].astype(jnp.int32)
    npad = EP - E
    srcp = jnp.concatenate([src, jnp.zeros((npad,), jnp.int32)])
    dstp = jnp.concatenate([dst, jnp.full((npad,), PADDST, jnp.int32)])
    src3 = srcp.reshape(NW, EPT)
    dst3 = dstp.reshape(NW, EPT)
    degp = _deg_sc(dstp).reshape(NW, NP)
    dinv = _dinv_tc(degp)
    g1 = _mm1_tc(x, W1, dinv)
    s1 = _scatter_sc(src3, dst3, g1)
    g2 = _mid_tc(s1, g1, dinv, b1.reshape(1, D), W2)
    s2 = _scatter_sc(src3, dst3, g2)
    return _fin_tc(s2, g2, dinv, b2.reshape(1, D))


# speedup vs baseline: 1.1401x; 1.1401x over previous
import functools

import jax
import jax.numpy as jnp
from jax import lax
from jax.experimental import pallas as pl
from jax.experimental.pallas import tpu as pltpu
from jax.experimental.pallas import tpu_sc as plsc

N = 10000
E = 320000
D = 128
NC = 2
NS = 16
NW = NC * NS
EPW = E // NW
CH = 80
NCHK = EPW // CH
EPTD = 10048
EPD = NW * EPTD
NP = 10240
PADDST = NP - 8
RPS = NP // NS

_mesh = plsc.VectorSubcoreMesh(core_axis_name="c", subcore_axis_name="s")
_sc_params = pltpu.CompilerParams(needs_layout_passes=False)

@functools.partial(
    pl.kernel,
    out_type=jax.ShapeDtypeStruct((NW * NP,), jnp.float32),
    mesh=_mesh,
    scratch_types=[
        pltpu.VMEM((NP,), jnp.float32),
        pltpu.VMEM((EPTD,), jnp.int32),
    ],
    compiler_params=_sc_params,
)
def _deg_sc(dst_hbm, out_hbm, hist, idx):
    c = lax.axis_index("c")
    s = lax.axis_index("s")
    wid = c * NS + s
    zeros16 = jnp.zeros((16,), jnp.float32)
    ones16 = jnp.ones((16,), jnp.float32)

    def zbody(i, _):
        hist[pl.ds(i * 16, 16)] = zeros16
        return ()

    lax.fori_loop(0, NP // 16, zbody, ())
    pltpu.sync_copy(dst_hbm.at[pl.ds(wid * EPTD, EPTD)], idx)

    def body(i, _):
        iv = idx[pl.ds(i * 16, 16)]
        plsc.addupdate_scatter(hist, [iv], ones16)
        return ()

    lax.fori_loop(0, EPTD // 16, body, ())
    pltpu.sync_copy(hist, out_hbm.at[pl.ds(wid * NP, NP)])


@functools.partial(
    pl.kernel,
    out_type=jax.ShapeDtypeStruct((NC, NP, D), jnp.float32),
    mesh=_mesh,
    scratch_types=[
        pltpu.VMEM((CH,), jnp.int32),
        pltpu.VMEM((CH,), jnp.int32),
        pltpu.VMEM((CH, D), jnp.float32),
        pltpu.VMEM_SHARED((NP, D), jnp.float32),
        pltpu.SemaphoreType.DMA,
    ],
    compiler_params=_sc_params,
)
def _scatter_sc(src_hbm, dst_hbm, g_hbm, out_hbm,
                isv, idv, rows, acc, sem):
    c = lax.axis_index("c")
    s = lax.axis_index("s")
    wid = c * NS + s
    base = wid * EPW

    zeros16 = jnp.zeros((16,), jnp.float32)

    def zrow(r, _):
        for jc in range(D // 16):
            rows[r, pl.ds(jc * 16, 16)] = zeros16
        return ()

    lax.fori_loop(0, CH, zrow, ())
    for j in range(RPS // CH):
        pltpu.sync_copy(rows, acc.at[pl.ds(s * RPS + j * CH, CH)])

    plsc.subcore_barrier()

    def body(i, _):
        pltpu.sync_copy(src_hbm.at[pl.ds(base + i * CH, CH)], isv)
        pltpu.sync_copy(dst_hbm.at[pl.ds(base + i * CH, CH)], idv)
        pltpu.async_copy(g_hbm.at[isv], rows, sem).wait()
        pltpu.sync_copy(rows, acc.at[idv], add=True)
        return ()

    lax.fori_loop(0, NCHK, body, ())
    plsc.subcore_barrier()
    pltpu.sync_copy(acc.at[pl.ds(s * RPS, RPS)],
                    out_hbm.at[c, pl.ds(s * RPS, RPS)])


BN = 400


def _dinv_body(p_ref, o_ref):
    deg = jnp.sum(p_ref[...], axis=0) + 1.0
    o_ref[...] = lax.rsqrt(deg)[:N, None]


_dinv_tc = pl.pallas_call(
    _dinv_body,
    out_shape=jax.ShapeDtypeStruct((N, 1), jnp.float32),
)


def _mm1_body(x_ref, w_ref, dv_ref, o_ref):
    h = jnp.dot(x_ref[...], w_ref[...], preferred_element_type=jnp.float32)
    o_ref[...] = h * dv_ref[...]


_mm1_tc = pl.pallas_call(
    _mm1_body,
    grid=(N // BN,),
    in_specs=[
        pl.BlockSpec((BN, D), lambda i: (i, 0)),
        pl.BlockSpec((D, D), lambda i: (0, 0)),
        pl.BlockSpec((BN, 1), lambda i: (i, 0)),
    ],
    out_specs=pl.BlockSpec((BN, D), lambda i: (i, 0)),
    out_shape=jax.ShapeDtypeStruct((N, D), jnp.float32),
)


def _mid_body(s_ref, g1_ref, dv_ref, b1_ref, w2_ref, o_ref):
    ssum = s_ref[0] + s_ref[1]
    agg = (ssum + g1_ref[...]) * dv_ref[...] + b1_ref[...]
    h1 = jnp.maximum(agg, 0.0)
    h2 = jnp.dot(h1, w2_ref[...], preferred_element_type=jnp.float32)
    o_ref[...] = h2 * dv_ref[...]


_mid_tc = pl.pallas_call(
    _mid_body,
    grid=(N // BN,),
    in_specs=[
        pl.BlockSpec((NC, BN, D), lambda i: (0, i, 0)),
        pl.BlockSpec((BN, D), lambda i: (i, 0)),
        pl.BlockSpec((BN, 1), lambda i: (i, 0)),
        pl.BlockSpec((1, D), lambda i: (0, 0)),
        pl.BlockSpec((D, D), lambda i: (0, 0)),
    ],
    out_specs=pl.BlockSpec((BN, D), lambda i: (i, 0)),
    out_shape=jax.ShapeDtypeStruct((N, D), jnp.float32),
)


def _fin_body(s_ref, g2_ref, dv_ref, b2_ref, o_ref):
    ssum = s_ref[0] + s_ref[1]
    o_ref[...] = (ssum + g2_ref[...]) * dv_ref[...] + b2_ref[...]


_fin_tc = pl.pallas_call(
    _fin_body,
    grid=(N // BN,),
    in_specs=[
        pl.BlockSpec((NC, BN, D), lambda i: (0, i, 0)),
        pl.BlockSpec((BN, D), lambda i: (i, 0)),
        pl.BlockSpec((BN, 1), lambda i: (i, 0)),
        pl.BlockSpec((1, D), lambda i: (0, 0)),
    ],
    out_specs=pl.BlockSpec((BN, D), lambda i: (i, 0)),
    out_shape=jax.ShapeDtypeStruct((N, D), jnp.float32),
)


def kernel(x, edge_index, W1, b1, W2, b2):
    src = edge_index[0].astype(jnp.int32)
    dst = edge_index[1].astype(jnp.int32)
    dstd = jnp.concatenate([dst, jnp.full((EPD - E,), PADDST, jnp.int32)])
    degp = _deg_sc(dstd).reshape(NW, NP)
    dinv = _dinv_tc(degp)
    g1 = _mm1_tc(x, W1, dinv)
    s1 = _scatter_sc(src, dst, g1)
    g2 = _mid_tc(s1, g1, dinv, b1.reshape(1, D), W2)
    s2 = _scatter_sc(src, dst, g2)
    return _fin_tc(s2, g2, dinv, b2.reshape(1, D))


# speedup vs baseline: 2.4206x; 2.1231x over previous
import functools

import jax
import jax.numpy as jnp
from jax import lax
from jax.experimental import pallas as pl
from jax.experimental.pallas import tpu as pltpu
from jax.experimental.pallas import tpu_sc as plsc

N = 10000
E = 320000
D = 128
NC = 2
NS = 16
NW = NC * NS
EPW = E // NW
CH = 80
NCHK = EPW // CH
EPTD = 10048
EPD = NW * EPTD
NP = 10240
PADDST = NP - 8
RPS = NP // NS

_mesh = plsc.VectorSubcoreMesh(core_axis_name="c", subcore_axis_name="s")
_sc_params = pltpu.CompilerParams(needs_layout_passes=False)

@functools.partial(
    pl.kernel,
    out_type=jax.ShapeDtypeStruct((NW * NP,), jnp.float32),
    mesh=_mesh,
    scratch_types=[
        pltpu.VMEM((NP,), jnp.float32),
        pltpu.VMEM((EPTD,), jnp.int32),
    ],
    compiler_params=_sc_params,
)
def _deg_sc(dst_hbm, out_hbm, hist, idx):
    c = lax.axis_index("c")
    s = lax.axis_index("s")
    wid = c * NS + s
    zeros16 = jnp.zeros((16,), jnp.float32)
    ones16 = jnp.ones((16,), jnp.float32)

    def zbody(i, _):
        hist[pl.ds(i * 16, 16)] = zeros16
        return ()

    lax.fori_loop(0, NP // 16, zbody, ())
    pltpu.sync_copy(dst_hbm.at[pl.ds(wid * EPTD, EPTD)], idx)

    def body(i, _):
        iv = idx[pl.ds(i * 16, 16)]
        plsc.addupdate_scatter(hist, [iv], ones16)
        return ()

    lax.fori_loop(0, EPTD // 16, body, ())
    pltpu.sync_copy(hist, out_hbm.at[pl.ds(wid * NP, NP)])


@functools.partial(
    pl.kernel,
    out_type=jax.ShapeDtypeStruct((NC, NP, D), jnp.float32),
    mesh=_mesh,
    scratch_types=[
        pltpu.VMEM((EPW,), jnp.int32),
        pltpu.VMEM((NCHK, CH), jnp.int32),
        pltpu.VMEM((CH, D), jnp.float32),
        pltpu.VMEM((CH, D), jnp.float32),
        pltpu.VMEM_SHARED((NP, D), jnp.float32),
        pltpu.SemaphoreType.DMA,
        pltpu.SemaphoreType.DMA,
    ],
    compiler_params=_sc_params,
)
def _scatter_sc(src_hbm, dst_hbm, g_hbm, out_hbm,
                isv, idv, rows_a, rows_b, acc, sem_a, sem_b):
    c = lax.axis_index("c")
    s = lax.axis_index("s")
    wid = c * NS + s

    pltpu.sync_copy(src_hbm.at[wid], isv)
    pltpu.sync_copy(dst_hbm.at[wid], idv)

    zeros16 = jnp.zeros((16,), jnp.float32)

    def zrow(r, _):
        for jc in range(D // 16):
            rows_a[r, pl.ds(jc * 16, 16)] = zeros16
        return ()

    lax.fori_loop(0, CH, zrow, ())
    for j in range(RPS // CH):
        pltpu.sync_copy(rows_a, acc.at[pl.ds(s * RPS + j * CH, CH)])

    plsc.subcore_barrier()

    def isl(i):
        return isv.at[pl.ds(i * CH, CH)]

    pltpu.async_copy(g_hbm.at[isl(0)], rows_a, sem_a)

    def body(i2, _):
        ia = 2 * i2
        ib = ia + 1
        pltpu.async_copy(g_hbm.at[isl(ib)], rows_b, sem_b)
        pltpu.make_async_copy(g_hbm.at[isl(ia)], rows_a, sem_a).wait()
        pltpu.sync_copy(rows_a, acc.at[idv.at[ia]], add=True)
        pltpu.async_copy(g_hbm.at[isl(ia + 2)], rows_a, sem_a)
        pltpu.make_async_copy(g_hbm.at[isl(ib)], rows_b, sem_b).wait()
        pltpu.sync_copy(rows_b, acc.at[idv.at[ib]], add=True)
        return ()

    lax.fori_loop(0, (NCHK - 1) // 2, body, ())
    pltpu.make_async_copy(g_hbm.at[isl(NCHK - 1)], rows_a, sem_a).wait()
    pltpu.sync_copy(rows_a, acc.at[idv.at[NCHK - 1]], add=True)
    plsc.subcore_barrier()
    pltpu.sync_copy(acc.at[pl.ds(s * RPS, RPS)],
                    out_hbm.at[c, pl.ds(s * RPS, RPS)])


BN = 400


def _dinv_body(p_ref, o_ref):
    deg = jnp.sum(p_ref[...], axis=0) + 1.0
    o_ref[...] = lax.rsqrt(deg)[:N, None]


_dinv_tc = pl.pallas_call(
    _dinv_body,
    out_shape=jax.ShapeDtypeStruct((N, 1), jnp.float32),
)


def _mm1_body(x_ref, w_ref, dv_ref, o_ref):
    h = jnp.dot(x_ref[...], w_ref[...], preferred_element_type=jnp.float32)
    o_ref[...] = h * dv_ref[...]


_mm1_tc = pl.pallas_call(
    _mm1_body,
    grid=(N // BN,),
    in_specs=[
        pl.BlockSpec((BN, D), lambda i: (i, 0)),
        pl.BlockSpec((D, D), lambda i: (0, 0)),
        pl.BlockSpec((BN, 1), lambda i: (i, 0)),
    ],
    out_specs=pl.BlockSpec((BN, D), lambda i: (i, 0)),
    out_shape=jax.ShapeDtypeStruct((N, D), jnp.float32),
)


def _mid_body(s_ref, g1_ref, dv_ref, b1_ref, w2_ref, o_ref):
    ssum = s_ref[0] + s_ref[1]
    agg = (ssum + g1_ref[...]) * dv_ref[...] + b1_ref[...]
    h1 = jnp.maximum(agg, 0.0)
    h2 = jnp.dot(h1, w2_ref[...], preferred_element_type=jnp.float32)
    o_ref[...] = h2 * dv_ref[...]


_mid_tc = pl.pallas_call(
    _mid_body,
    grid=(N // BN,),
    in_specs=[
        pl.BlockSpec((NC, BN, D), lambda i: (0, i, 0)),
        pl.BlockSpec((BN, D), lambda i: (i, 0)),
        pl.BlockSpec((BN, 1), lambda i: (i, 0)),
        pl.BlockSpec((1, D), lambda i: (0, 0)),
        pl.BlockSpec((D, D), lambda i: (0, 0)),
    ],
    out_specs=pl.BlockSpec((BN, D), lambda i: (i, 0)),
    out_shape=jax.ShapeDtypeStruct((N, D), jnp.float32),
)


def _fin_body(s_ref, g2_ref, dv_ref, b2_ref, o_ref):
    ssum = s_ref[0] + s_ref[1]
    o_ref[...] = (ssum + g2_ref[...]) * dv_ref[...] + b2_ref[...]


_fin_tc = pl.pallas_call(
    _fin_body,
    grid=(N // BN,),
    in_specs=[
        pl.BlockSpec((NC, BN, D), lambda i: (0, i, 0)),
        pl.BlockSpec((BN, D), lambda i: (i, 0)),
        pl.BlockSpec((BN, 1), lambda i: (i, 0)),
        pl.BlockSpec((1, D), lambda i: (0, 0)),
    ],
    out_specs=pl.BlockSpec((BN, D), lambda i: (i, 0)),
    out_shape=jax.ShapeDtypeStruct((N, D), jnp.float32),
)


def kernel(x, edge_index, W1, b1, W2, b2):
    src = edge_index[0].astype(jnp.int32)
    dst = edge_index[1].astype(jnp.int32)
    dstd = jnp.concatenate([dst, jnp.full((EPD - E,), PADDST, jnp.int32)])
    srcr = src.reshape(NW, EPW)
    dstr = dst.reshape(NW, NCHK, CH)
    degp = _deg_sc(dstd).reshape(NW, NP)
    dinv = _dinv_tc(degp)
    g1 = _mm1_tc(x, W1, dinv)
    s1 = _scatter_sc(srcr, dstr, g1)
    g2 = _mid_tc(s1, g1, dinv, b1.reshape(1, D), W2)
    s2 = _scatter_sc(srcr, dstr, g2)
    return _fin_tc(s2, g2, dinv, b2.reshape(1, D))
